# Initial kernel scaffold; baseline (speedup 1.0000x reference)
import jax
import jax.numpy as jnp
from jax.experimental import pallas as pl


def kernel(posterior_prob, latent_var_inds, latent_neg_mask, observed_neg_mask, observed_rule_cnts, rule_weights):
    raise NotImplementedError("write your pallas kernel here")


# speedup vs baseline: 3.0752x; 3.0752x over previous
import functools

import jax
import jax.numpy as jnp
from jax import lax
from jax.experimental import pallas as pl
from jax.experimental.pallas import tpu as pltpu
from jax.experimental.pallas import tpu_sc as plsc

N_ATOMS = 1000000
R = 8
G = 100000
L = 3
O = 4

NC = 2
NS = 16
NW = NC * NS
GP = 100352
TP = (R * GP) // NW
K = 3136
NCH = TP // K
NGRP = K // 16
K3 = 3 * K
K4 = 4 * K


def _sc_body(tbl_hbm, idx_hbm, m_hbm, obs_hbm, part_hbm,
             idx_v, p_v, m_v, obs_v, acc_v, sem):
    wid = lax.axis_index("s") * NC + lax.axis_index("c")
    g0_3 = wid * (TP * 3)
    g0_4 = wid * (TP * 4)

    iota = lax.iota(jnp.int32, 16)
    i3 = iota * 3
    i4 = iota * 4
    zero_v = jnp.zeros((16,), jnp.float32)
    one_v = jnp.ones((16,), jnp.float32)

    def chunk_body(ci, acc):
        b3 = g0_3 + ci * K3
        b4 = g0_4 + ci * K4
        pltpu.sync_copy(idx_hbm.at[pl.ds(b3, K3)], idx_v)
        pltpu.async_copy(tbl_hbm.at[idx_v], p_v, sem).wait()
        pltpu.sync_copy(m_hbm.at[pl.ds(b3, K3)], m_v)
        pltpu.sync_copy(obs_hbm.at[pl.ds(b4, K4)], obs_v)

        def grp(i, a):
            base3 = i * 48 + i3
            base4 = i * 64 + i4
            prod = one_v
            for l in range(L):
                ml = plsc.load_gather(m_v, [base3 + l])
                pv = plsc.load_gather(p_v, [base3 + l])
                sel = jnp.where(ml == 1, pv, 1.0 - pv)
                prod = prod * sel
            ssum = plsc.load_gather(obs_v, [base4])
            for o in range(1, O):
                ssum = ssum + plsc.load_gather(obs_v, [base4 + o])
            return a + jnp.where(ssum == 0, prod, zero_v)

        return lax.fori_loop(0, NGRP, grp, acc)

    acc = lax.fori_loop(0, NCH, chunk_body, zero_v)
    acc_v[...] = acc
    pltpu.sync_copy(acc_v, part_hbm.at[wid])


_sc_kernel = pl.kernel(
    _sc_body,
    out_type=jax.ShapeDtypeStruct((NW, 16), jnp.float32),
    mesh=plsc.VectorSubcoreMesh(core_axis_name="c", subcore_axis_name="s"),
    compiler_params=pltpu.CompilerParams(needs_layout_passes=False),
    scratch_types=[
        pltpu.VMEM((K3,), jnp.int32),
        pltpu.VMEM((K3,), jnp.float32),
        pltpu.VMEM((K3,), jnp.int32),
        pltpu.VMEM((K4,), jnp.int32),
        pltpu.VMEM((16,), jnp.float32),
        pltpu.SemaphoreType.DMA,
    ],
)


def _finish_body(part_ref, wrow_ref, cnt_ref, w_ref, out_ref):
    c0 = jnp.sum(w_ref[...] * (jnp.float32(G) + cnt_ref[...]))
    s = jnp.sum(part_ref[...] * wrow_ref[...])
    out_ref[...] = jnp.reshape(c0 - s, (1, 1))


_finish = pl.pallas_call(
    _finish_body,
    out_shape=jax.ShapeDtypeStruct((1, 1), jnp.float32),
)


def kernel(posterior_prob, latent_var_inds, latent_neg_mask, observed_neg_mask,
           observed_rule_cnts, rule_weights):
    pad = GP - G
    idx_flat = jnp.pad(latent_var_inds, ((0, 0), (0, pad), (0, 0))).reshape(-1)
    m_flat = jnp.pad(latent_neg_mask, ((0, 0), (0, pad), (0, 0))).reshape(-1)
    obs_flat = jnp.pad(observed_neg_mask, ((0, 0), (0, pad), (0, 0)),
                       constant_values=1).reshape(-1)

    partials = _sc_kernel(posterior_prob, idx_flat, m_flat, obs_flat)

    wrow = jnp.repeat(rule_weights[0], NW // R).reshape(NW, 1)
    out = _finish(partials, wrow, observed_rule_cnts.reshape(1, R),
                  rule_weights)
    return out.reshape(1)


# speedup vs baseline: 3.4407x; 1.1188x over previous
import jax
import jax.numpy as jnp
from jax import lax
from jax.experimental import pallas as pl
from jax.experimental.pallas import tpu as pltpu
from jax.experimental.pallas import tpu_sc as plsc

N_ATOMS = 1000000
R = 8
G = 100000
L = 3
O = 4

NC = 2
NS = 16
NW = NC * NS
T = (R * G) // NW
K = 3120
NCH = 8
NGRP = K // 16
K3 = 3 * K
K4 = 4 * K
TAIL = T - NCH * K
TAIL3 = 3 * TAIL
TAIL4 = 4 * TAIL
TGRP = 3
TB3 = TGRP * 48
TB4 = TGRP * 64


def _sc_body(tbl_hbm, idx_hbm, m_hbm, obs_hbm, part_hbm,
             idx_v, p_v, m_v, obs_v,
             idxt_v, pt_v, mt_v, obst_v, acc_v, sem):
    wid = lax.axis_index("s") * NC + lax.axis_index("c")
    g0_3 = wid * (T * 3)
    g0_4 = wid * (T * 4)

    iota = lax.iota(jnp.int32, 16)
    i3 = iota * 3
    i4 = iota * 4
    zero_v = jnp.zeros((16,), jnp.float32)
    one_v = jnp.ones((16,), jnp.float32)

    idxt_v[pl.ds(TAIL3, 16)] = jnp.zeros((16,), jnp.int32)
    idxt_v[pl.ds(TB3 - 16, 16)] = jnp.zeros((16,), jnp.int32)

    def group_term(pv_ref, mv_ref, ov_ref, i):
        base3 = i * 48 + i3
        base4 = i * 64 + i4
        prod = one_v
        for l in range(L):
            ml = plsc.load_gather(mv_ref, [base3 + l])
            pv = plsc.load_gather(pv_ref, [base3 + l])
            sel = jnp.where(ml == 1, pv, 1.0 - pv)
            prod = prod * sel
        ssum = plsc.load_gather(ov_ref, [base4])
        for o in range(1, O):
            ssum = ssum + plsc.load_gather(ov_ref, [base4 + o])
        return prod, ssum

    def chunk_body(ci, acc):
        b3 = g0_3 + ci * K3
        b4 = g0_4 + ci * K4
        pltpu.sync_copy(idx_hbm.at[pl.ds(b3, K3)], idx_v)
        pltpu.async_copy(tbl_hbm.at[idx_v], p_v, sem).wait()
        pltpu.sync_copy(m_hbm.at[pl.ds(b3, K3)], m_v)
        pltpu.sync_copy(obs_hbm.at[pl.ds(b4, K4)], obs_v)

        def grp(i, a):
            prod, ssum = group_term(p_v, m_v, obs_v, i)
            return a + jnp.where(ssum == 0, prod, zero_v)

        return lax.fori_loop(0, NGRP, grp, acc)

    acc = lax.fori_loop(0, NCH, chunk_body, zero_v)

    b3 = g0_3 + NCH * K3
    b4 = g0_4 + NCH * K4
    pltpu.sync_copy(idx_hbm.at[pl.ds(b3, TAIL3)], idxt_v.at[pl.ds(0, TAIL3)])
    pltpu.async_copy(tbl_hbm.at[idxt_v], pt_v, sem).wait()
    pltpu.sync_copy(m_hbm.at[pl.ds(b3, TAIL3)], mt_v.at[pl.ds(0, TAIL3)])
    pltpu.sync_copy(obs_hbm.at[pl.ds(b4, TAIL4)], obst_v.at[pl.ds(0, TAIL4)])
    for j in range(TGRP):
        prod, ssum = group_term(pt_v, mt_v, obst_v, j)
        valid = (j * 16 + iota) < TAIL
        acc = acc + jnp.where((ssum == 0) & valid, prod, zero_v)

    acc_v[...] = acc
    pltpu.sync_copy(acc_v, part_hbm.at[wid])


_sc_kernel = pl.kernel(
    _sc_body,
    out_type=jax.ShapeDtypeStruct((NW, 16), jnp.float32),
    mesh=plsc.VectorSubcoreMesh(core_axis_name="c", subcore_axis_name="s"),
    compiler_params=pltpu.CompilerParams(needs_layout_passes=False),
    scratch_types=[
        pltpu.VMEM((K3,), jnp.int32),
        pltpu.VMEM((K3,), jnp.float32),
        pltpu.VMEM((K3,), jnp.int32),
        pltpu.VMEM((K4,), jnp.int32),
        pltpu.VMEM((TB3,), jnp.int32),
        pltpu.VMEM((TB3,), jnp.float32),
        pltpu.VMEM((TB3,), jnp.int32),
        pltpu.VMEM((TB4,), jnp.int32),
        pltpu.VMEM((16,), jnp.float32),
        pltpu.SemaphoreType.DMA,
    ],
)


def _finish_body(part_ref, wrow_ref, cnt_ref, w_ref, out_ref):
    c0 = jnp.sum(w_ref[...] * (jnp.float32(G) + cnt_ref[...]))
    s = jnp.sum(part_ref[...] * wrow_ref[...])
    out_ref[...] = jnp.reshape(c0 - s, (1, 1))


_finish = pl.pallas_call(
    _finish_body,
    out_shape=jax.ShapeDtypeStruct((1, 1), jnp.float32),
)


def kernel(posterior_prob, latent_var_inds, latent_neg_mask, observed_neg_mask,
           observed_rule_cnts, rule_weights):
    idx_flat = latent_var_inds.reshape(-1)
    m_flat = latent_neg_mask.reshape(-1)
    obs_flat = observed_neg_mask.reshape(-1)

    partials = _sc_kernel(posterior_prob, idx_flat, m_flat, obs_flat)

    wrow = jnp.repeat(rule_weights[0], NW // R).reshape(NW, 1)
    out = _finish(partials, wrow, observed_rule_cnts.reshape(1, R),
                  rule_weights)
    return out.reshape(1)


# speedup vs baseline: 99.7024x; 28.9776x over previous
import jax
import jax.numpy as jnp
from jax import lax
from jax.experimental import pallas as pl
from jax.experimental.pallas import tpu as pltpu
from jax.experimental.pallas import tpu_sc as plsc

N_ATOMS = 1000000
R = 8
G = 100000
L = 3
O = 4

NC = 2
NS = 16
NW = NC * NS
T = (R * G) // NW
K = 6240
NCH = 4
NGRP = K // 16
TAIL = T - NCH * K
TGRP = 3
TB = TGRP * 16
FLAG = 1 << 21
IDXMASK = FLAG - 1


def _sc_body(tbl_hbm, pk0_hbm, pk1_hbm, pk2_hbm, part_hbm,
             pk0_v, pk1_v, pk2_v, idx_v, p_v,
             pkt0_v, pkt1_v, pkt2_v, idxt_v, pt_v, acc_v, sem):
    wid = lax.axis_index("s") * NC + lax.axis_index("c")
    g0 = wid * T

    iota = lax.iota(jnp.int32, 16)
    zero_v = jnp.zeros((16,), jnp.float32)
    zero_i = jnp.zeros((16,), jnp.int32)
    one_v = jnp.ones((16,), jnp.float32)

    pkt0_v[pl.ds(TB - 16, 16)] = zero_i
    pkt1_v[pl.ds(TB - 16, 16)] = zero_i
    pkt2_v[pl.ds(TB - 16, 16)] = zero_i

    def unpack_store(pk_refs, idx_ref, n, i):
        ds = pl.ds(i * 16, 16)
        for l in range(L):
            v = pk_refs[l][ds]
            idx_ref[pl.ds(l * n + i * 16, 16)] = (
                lax.shift_right_logical(v & IDXMASK, 1))

    def accum_term(pk_refs, p_ref, n, i):
        ds = pl.ds(i * 16, 16)
        prod = one_v
        v0 = pk_refs[0][ds]
        for l in range(L):
            v = pk_refs[l][ds] if l else v0
            pv = p_ref[pl.ds(l * n + i * 16, 16)]
            sel = jnp.where((v & 1) == 1, pv, 1.0 - pv)
            prod = prod * sel
        return prod, v0 < FLAG

    def chunk_body(ci, acc):
        b = g0 + ci * K
        pltpu.sync_copy(pk0_hbm.at[pl.ds(b, K)], pk0_v)
        pltpu.sync_copy(pk1_hbm.at[pl.ds(b, K)], pk1_v)
        pltpu.sync_copy(pk2_hbm.at[pl.ds(b, K)], pk2_v)

        def pass1(i, carry):
            unpack_store((pk0_v, pk1_v, pk2_v), idx_v, K, i)
            return carry
        lax.fori_loop(0, NGRP, pass1, 0)

        pltpu.async_copy(tbl_hbm.at[idx_v], p_v, sem).wait()

        def pass2(i, a):
            prod, obs_zero = accum_term((pk0_v, pk1_v, pk2_v), p_v, K, i)
            return a + jnp.where(obs_zero, prod, zero_v)
        return lax.fori_loop(0, NGRP, pass2, acc)

    acc = lax.fori_loop(0, NCH, chunk_body, zero_v)

    b = g0 + NCH * K
    pltpu.sync_copy(pk0_hbm.at[pl.ds(b, TAIL)], pkt0_v.at[pl.ds(0, TAIL)])
    pltpu.sync_copy(pk1_hbm.at[pl.ds(b, TAIL)], pkt1_v.at[pl.ds(0, TAIL)])
    pltpu.sync_copy(pk2_hbm.at[pl.ds(b, TAIL)], pkt2_v.at[pl.ds(0, TAIL)])
    for j in range(TGRP):
        unpack_store((pkt0_v, pkt1_v, pkt2_v), idxt_v, TB, j)
    pltpu.async_copy(tbl_hbm.at[idxt_v], pt_v, sem).wait()
    for j in range(TGRP):
        prod, obs_zero = accum_term((pkt0_v, pkt1_v, pkt2_v), pt_v, TB, j)
        valid = (j * 16 + iota) < TAIL
        acc = acc + jnp.where(obs_zero & valid, prod, zero_v)

    acc_v[...] = acc
    pltpu.sync_copy(acc_v, part_hbm.at[wid])


_sc_kernel = pl.kernel(
    _sc_body,
    out_type=jax.ShapeDtypeStruct((NW, 16), jnp.float32),
    mesh=plsc.VectorSubcoreMesh(core_axis_name="c", subcore_axis_name="s"),
    compiler_params=pltpu.CompilerParams(needs_layout_passes=False),
    scratch_types=[
        pltpu.VMEM((K,), jnp.int32),
        pltpu.VMEM((K,), jnp.int32),
        pltpu.VMEM((K,), jnp.int32),
        pltpu.VMEM((L * K,), jnp.int32),
        pltpu.VMEM((L * K,), jnp.float32),
        pltpu.VMEM((TB,), jnp.int32),
        pltpu.VMEM((TB,), jnp.int32),
        pltpu.VMEM((TB,), jnp.int32),
        pltpu.VMEM((L * TB,), jnp.int32),
        pltpu.VMEM((L * TB,), jnp.float32),
        pltpu.VMEM((16,), jnp.float32),
        pltpu.SemaphoreType.DMA,
    ],
)


def _finish_body(part_ref, wrow_ref, cnt_ref, w_ref, out_ref):
    c0 = jnp.sum(w_ref[...] * (jnp.float32(G) + cnt_ref[...]))
    s = jnp.sum(part_ref[...] * wrow_ref[...])
    out_ref[...] = jnp.reshape(c0 - s, (1, 1))


_finish = pl.pallas_call(
    _finish_body,
    out_shape=jax.ShapeDtypeStruct((1, 1), jnp.float32),
)


def kernel(posterior_prob, latent_var_inds, latent_neg_mask, observed_neg_mask,
           observed_rule_cnts, rule_weights):
    base = latent_var_inds * 2 + latent_neg_mask
    obs_nz = jnp.sum(observed_neg_mask, axis=-1) != 0
    pk0 = (base[:, :, 0] + jnp.where(obs_nz, FLAG, 0)).reshape(-1)
    pk1 = base[:, :, 1].reshape(-1)
    pk2 = base[:, :, 2].reshape(-1)

    partials = _sc_kernel(posterior_prob, pk0, pk1, pk2)

    wrow = jnp.repeat(rule_weights[0], NW // R).reshape(NW, 1)
    out = _finish(partials, wrow, observed_rule_cnts.reshape(1, R),
                  rule_weights)
    return out.reshape(1)
